# baseline (speedup 1.0000x reference)
import functools
import numpy as np
import jax
import jax.numpy as jnp
from jax import lax
from jax.experimental import pallas as pl
from jax.experimental.pallas import tpu as pltpu
from jax.experimental.pallas import tpu_sc as plsc

FEAT_STRIDE = 16
_SCALES = np.array([8.0, 16.0, 32.0])
_RATIOS = np.array([0.5, 1.0, 2.0])


def _gen_anchors_np():
    base = np.array([1, 1, 16, 16], dtype=np.float64) - 1
    w = base[2] - base[0] + 1
    h = base[3] - base[1] + 1
    x_ctr = base[0] + 0.5 * (w - 1)
    y_ctr = base[1] + 0.5 * (h - 1)
    size = w * h
    size_ratios = size / _RATIOS
    ws0 = np.round(np.sqrt(size_ratios))
    hs0 = np.round(ws0 * _RATIOS)
    ratio_anchors = np.hstack((
        x_ctr - 0.5 * (ws0[:, None] - 1), y_ctr - 0.5 * (hs0[:, None] - 1),
        x_ctr + 0.5 * (ws0[:, None] - 1), y_ctr + 0.5 * (hs0[:, None] - 1)))
    outs = []
    for i in range(ratio_anchors.shape[0]):
        a = ratio_anchors[i]
        aw = a[2] - a[0] + 1
        ah = a[3] - a[1] + 1
        ax = a[0] + 0.5 * (aw - 1)
        ay = a[1] + 0.5 * (ah - 1)
        ws = aw * _SCALES
        hs = ah * _SCALES
        outs.append(np.hstack((
            ax - 0.5 * (ws[:, None] - 1), ay - 0.5 * (hs[:, None] - 1),
            ax + 0.5 * (ws[:, None] - 1), ay + 0.5 * (hs[:, None] - 1))))
    return np.vstack(outs)


_FEAT_H, _FEAT_W = 75, 120
_A = 9
_N_TRUE = _FEAT_H * _FEAT_W * _A
_NW = 32
_B, _KGT, _R = 4, 50, 5
_KP = 64
_NG = _B * _KP
_GPW = _NG // _NW
_NRPN = _B * 2 * _A * _FEAT_H * _FEAT_W

_A0 = _gen_anchors_np()
_CX1 = [float(np.float32(_A0[a, 0])) for a in range(_A)]
_CY1 = [float(np.float32(_A0[a, 1])) for a in range(_A)]
_CX2P = [float(np.float32(_A0[a, 2] + 1.0)) for a in range(_A)]
_CY2P = [float(np.float32(_A0[a, 3] + 1.0)) for a in range(_A)]
_CAREA = [float(np.float32((_A0[a, 2] - _A0[a, 0] + 1.0) *
                           (_A0[a, 3] - _A0[a, 1] + 1.0))) for a in range(_A)]
_CKLO = [int(np.ceil(-_A0[a, 0] / 16.0)) for a in range(_A)]
_RKLO = [int(np.ceil(-_A0[a, 1] / 16.0)) for a in range(_A)]

_REAL = [b * _KP + k for b in range(_B) for k in range(_KGT)]
_PADS = [b * _KP + k for b in range(_B) for k in range(_KGT, _KP)]
_SCHED = [[] for _ in range(_NW)]
for _i, _s in enumerate(_REAL):
    _SCHED[_i % _NW].append(_s)
_pit = iter(_PADS)
for _wl in _SCHED:
    while len(_wl) < _GPW:
        _wl.append(next(_pit))
_PERM = np.array([s for wl in _SCHED for s in wl], dtype=np.int64)
_INVPERM = np.empty(_NG, dtype=np.int64)
_INVPERM[_PERM] = np.arange(_NG)


def _run(gtb, lim, rpnflat, cc2, sv16, sp16, boff, gtflat):
    mesh = plsc.VectorSubcoreMesh(core_axis_name="c", subcore_axis_name="s")

    @functools.partial(
        pl.kernel,
        out_type=jax.ShapeDtypeStruct((_NG * 5,), jnp.float32),
        mesh=mesh,
        scratch_types=[
            pltpu.VMEM((5, _GPW, 16), jnp.float32),
            pltpu.VMEM((32,), jnp.float32),
            pltpu.VMEM((5, 16), jnp.float32),
            pltpu.VMEM((5, 16), jnp.float32),
            pltpu.VMEM((5, 16), jnp.float32),
            pltpu.VMEM((16,), jnp.int32),
            pltpu.VMEM((80,), jnp.float32),
            pltpu.VMEM((16,), jnp.float32),
            pltpu.VMEM((_GPW * 5,), jnp.float32),
            pltpu.SemaphoreType.DMA,
        ],
        compiler_params=pltpu.CompilerParams(use_tc_tiling_on_sc=False,
                                             needs_layout_passes=False),
    )
    def k(gtb_h, lim_h, rpn_h, cc_h, sv_h, sp_h, boff_h, gtf_h, out_h,
          gt_v, lim_v, ccv, svv, spv, bv, gtv, pbuf, obuf, sem):
        wid = lax.axis_index("s") * 2 + lax.axis_index("c")
        sbase = wid * _GPW
        pltpu.sync_copy(gtb_h.at[:, pl.ds(sbase, _GPW), :], gt_v)
        pltpu.sync_copy(lim_h, lim_v)
        for r in range(5):
            pltpu.sync_copy(cc_h.at[r, pl.ds(sbase, 16)], ccv.at[r])
        pltpu.sync_copy(sv_h, svv)
        pltpu.sync_copy(sp_h, spv)
        pltpu.sync_copy(boff_h.at[pl.ds(sbase, 16)], bv)
        pltpu.sync_copy(gtf_h.at[pl.ds(sbase * 5, _GPW * 5)], gtv.at[pl.ds(0, _GPW * 5)])

        wp_s = lax.reduce_max(lim_v[pl.ds(0, 16)], (0,))
        hp_s = lax.reduce_max(lim_v[pl.ds(16, 16)], (0,))
        inv16s = jnp.float32(0.0625)
        ckhi_l = [((wp_s - jnp.float32(_CX2P[a] + 1.0)) * inv16s)
                  .astype(jnp.int32) for a in range(_A)]
        rkhi_l = [((hp_s - jnp.float32(_CY2P[a] + 1.0)) * inv16s)
                  .astype(jnp.int32) for a in range(_A)]
        iota_i = lax.iota(jnp.int32, 16)
        iota_f = iota_i.astype(jnp.float32)
        i16f = iota_f * jnp.float32(16.0)
        i9 = iota_i * 9
        zerov = jnp.zeros((16,), jnp.float32)
        imaxv = jnp.full((16,), jnp.int32(2**31 - 1), jnp.int32)
        f16 = jnp.float32(16.0)
        inv16 = jnp.float32(0.0625)

        def per_gt(t, argacc):
            g1 = gt_v[0, t, :]
            g2 = gt_v[1, t, :]
            g3 = gt_v[2, t, :]
            g4 = gt_v[3, t, :]
            g5 = gt_v[4, t, :]
            sx1 = lax.reduce_max(g1, (0,))
            sy1 = lax.reduce_max(g2, (0,))
            sx2p = lax.reduce_max(g3, (0,))
            sy2p = lax.reduce_max(g4, (0,))

            gm = jnp.float32(-1.0)
            gmi = jnp.int32(0)
            for a in range(_A):
                cx1 = jnp.float32(_CX1[a])
                cy1 = jnp.float32(_CY1[a])
                cx2p = jnp.float32(_CX2P[a])
                cy2p = jnp.float32(_CY2P[a])
                dxw = jnp.full((16,), jnp.float32(_CX2P[a] - _CX1[a]),
                               jnp.float32)
                areag5 = jnp.full((16,), jnp.float32(_CAREA[a]),
                                  jnp.float32) + g5
                wlo = jnp.maximum(((sx1 - cx2p) * inv16).astype(jnp.int32),
                                  _CKLO[a])
                whi = jnp.minimum(
                    ((sx2p - cx1) * inv16).astype(jnp.int32) + 1, ckhi_l[a])
                hlo = jnp.maximum(((sy1 - cy2p) * inv16).astype(jnp.int32),
                                  _RKLO[a])
                hhi = jnp.minimum(
                    ((sy2p - cy1) * inv16).astype(jnp.int32) + 1, rkhi_l[a])
                ncol = lax.shift_right_logical(jnp.maximum(whi - wlo, 0), 4) + 1
                whiv = jax.lax.broadcast(whi, (16,))

                def row_body(h, carry2):
                    hf = h.astype(jnp.float32)
                    ay1 = hf * f16 + cy1
                    ay2p = hf * f16 + cy2p
                    ihs = jnp.maximum(
                        jnp.minimum(ay2p, sy2p) - jnp.maximum(ay1, sy1),
                        jnp.float32(0.0))
                    ihbc = jax.lax.broadcast(ihs, (16,))
                    rowb9 = h * 1080 + a

                    def col_body(j, carry3):
                        bb, ba = carry3
                        col0 = wlo + j * 16
                        c0f = col0.astype(jnp.float32)
                        x1v = jax.lax.broadcast(c0f * f16 + cx1, (16,)) + i16f
                        x2pv = x1v + dxw
                        iw = jnp.minimum(x2pv, g3) - jnp.maximum(x1v, g1)
                        iwc = jnp.maximum(iw, zerov)
                        inter = iwc * ihbc
                        ua = areag5 - inter
                        v = inter / ua
                        coliv = jax.lax.broadcast(col0, (16,)) + iota_i
                        v = jnp.where(coliv <= whiv, v, zerov)
                        idxv = jax.lax.broadcast(col0 * 9 + rowb9, (16,)) + i9
                        gtm = v > bb
                        bb = jnp.where(gtm, v, bb)
                        ba = jnp.where(gtm, idxv, ba)
                        return bb, ba

                    return lax.fori_loop(0, ncol, col_body, carry2)

                best, barg = lax.fori_loop(
                    hlo, hhi + 1, row_body,
                    (jnp.full((16,), -1.0, jnp.float32),
                     jnp.zeros((16,), jnp.int32)))
                m_a = lax.reduce_max(best, (0,))
                cand = jnp.where(best == jax.lax.broadcast(m_a, (16,)),
                                 barg, imaxv)
                mi_a = lax.reduce_min(cand, (0,))
                upd = (m_a > gm) | ((m_a == gm) & (mi_a < gmi))
                gm = jnp.where(upd, m_a, gm)
                gmi = jnp.where(upd, mi_a, gmi)

            argacc = jnp.where(iota_i == jax.lax.broadcast(t, (16,)),
                               jax.lax.broadcast(gmi, (16,)), argacc)
            return argacc

        argacc = lax.fori_loop(0, _GPW, per_gt, jnp.zeros((16,), jnp.int32))

        gidx = jnp.minimum(argacc + bv[...], jnp.int32(_NRPN - 1))
        pltpu.async_copy(rpn_h.at[gidx], pbuf, sem).wait()
        p = pbuf[...]
        one = jnp.ones((16,), jnp.float32)
        aprod = jnp.ones((16,), jnp.float32)
        bprod = jnp.ones((16,), jnp.float32)
        for r in range(5):
            ccr = ccv[r, :]
            s = svv[r, :]
            sp = spv[r, :]
            hit = ccr == one
            aprod = aprod * jnp.where(hit, s, one - s)
            bprod = bprod * jnp.where(hit, one - sp, sp)
        ap = aprod * p
        bq = bprod * (one - p)
        mu = ap / ((ap + bq) + jnp.float32(1e-12))
        bg = mu < jnp.float32(0.5)
        lanemask = iota_i < _GPW
        for c in range(5):
            idxc = iota_i * 5 + c
            gvals = plsc.load_gather(gtv, [idxc], mask=lanemask)
            plsc.store_scatter(obuf, [idxc], jnp.where(bg, zerov, gvals),
                               mask=lanemask)
        pltpu.sync_copy(obuf, out_h.at[pl.ds(sbase * 5, _GPW * 5)])

    return k(gtb, lim, rpnflat, cc2, sv16, sp16, boff, gtflat)


def kernel(rpn_cls_prob, gt_boxes, num_boxes, im_info, crowdsourced_classes,
           alpha_con):
    del num_boxes
    gtp = jnp.pad(gt_boxes, ((0, 0), (0, _KP - _KGT), (0, 0)))
    gtp = gtp.at[:, _KGT:, 0:2].set(1.0e6)
    gtp = gtp.at[:, _KGT:, 2:4].set(1.0e6 + 19.0)
    gtp = gtp.reshape(_NG, 5)[_PERM]
    gx1 = gtp[:, 0]
    gy1 = gtp[:, 1]
    gx2p = gtp[:, 2] + 1.0
    gy2p = gtp[:, 3] + 1.0
    garea = (gtp[:, 2] - gtp[:, 0] + 1.0) * \
            (gtp[:, 3] - gtp[:, 1] + 1.0)
    gtb = jnp.stack([gx1, gy1, gx2p, gy2p, garea])
    gtb = jnp.broadcast_to(gtb[:, :, None], (5, _NG, 16))
    gtb = jnp.asarray(gtb, jnp.float32)

    wp = im_info[0, 1] + 1.0
    hp = im_info[0, 0] + 1.0
    lim = jnp.concatenate([jnp.full((16,), wp, jnp.float32),
                           jnp.full((16,), hp, jnp.float32)])

    rpnflat = rpn_cls_prob.reshape(-1)
    ccp = jnp.pad(crowdsourced_classes, ((0, 0), (0, _KP - _KGT), (0, 0)))
    cc2 = jnp.transpose(ccp, (2, 0, 1)).reshape(_R, _NG)[:, _PERM]
    cc2 = jnp.pad(cc2, ((0, 0), (0, 16)))
    asum = jnp.sum(alpha_con, axis=2, keepdims=True)
    alpha = alpha_con / asum
    sens = alpha[:, 1, 1]
    spec = alpha[:, 0, 0]
    sv16 = jnp.broadcast_to(sens[:, None], (_R, 16))
    sp16 = jnp.broadcast_to(spec[:, None], (_R, 16))
    boff = jnp.repeat(
        jnp.arange(_B, dtype=jnp.int32) * (2 * _A * _FEAT_H * _FEAT_W) +
        jnp.int32(_A * _FEAT_H * _FEAT_W), _KP)[_PERM]
    boff = jnp.pad(boff, (0, 16))
    gtflat = gtp.reshape(-1)

    outflat = _run(gtb, lim, rpnflat, cc2, sv16, sp16, boff, gtflat)
    out = outflat.reshape(_NG, 5)[_INVPERM]
    return out.reshape(_B, _KP, 5)[:, :_KGT, :]


# speedup vs baseline: 1.0093x; 1.0093x over previous
import functools
import numpy as np
import jax
import jax.numpy as jnp
from jax import lax
from jax.experimental import pallas as pl
from jax.experimental.pallas import tpu as pltpu
from jax.experimental.pallas import tpu_sc as plsc

FEAT_STRIDE = 16
_SCALES = np.array([8.0, 16.0, 32.0])
_RATIOS = np.array([0.5, 1.0, 2.0])


def _gen_anchors_np():
    base = np.array([1, 1, 16, 16], dtype=np.float64) - 1
    w = base[2] - base[0] + 1
    h = base[3] - base[1] + 1
    x_ctr = base[0] + 0.5 * (w - 1)
    y_ctr = base[1] + 0.5 * (h - 1)
    size = w * h
    size_ratios = size / _RATIOS
    ws0 = np.round(np.sqrt(size_ratios))
    hs0 = np.round(ws0 * _RATIOS)
    ratio_anchors = np.hstack((
        x_ctr - 0.5 * (ws0[:, None] - 1), y_ctr - 0.5 * (hs0[:, None] - 1),
        x_ctr + 0.5 * (ws0[:, None] - 1), y_ctr + 0.5 * (hs0[:, None] - 1)))
    outs = []
    for i in range(ratio_anchors.shape[0]):
        a = ratio_anchors[i]
        aw = a[2] - a[0] + 1
        ah = a[3] - a[1] + 1
        ax = a[0] + 0.5 * (aw - 1)
        ay = a[1] + 0.5 * (ah - 1)
        ws = aw * _SCALES
        hs = ah * _SCALES
        outs.append(np.hstack((
            ax - 0.5 * (ws[:, None] - 1), ay - 0.5 * (hs[:, None] - 1),
            ax + 0.5 * (ws[:, None] - 1), ay + 0.5 * (hs[:, None] - 1))))
    return np.vstack(outs)


_FEAT_H, _FEAT_W = 75, 120
_A = 9
_N_TRUE = _FEAT_H * _FEAT_W * _A
_NW = 32
_B, _KGT, _R = 4, 50, 5
_KP = 64
_NG = _B * _KP
_GPW = _NG // _NW
_NRPN = _B * 2 * _A * _FEAT_H * _FEAT_W

_A0 = _gen_anchors_np()
_CX1 = [float(np.float32(_A0[a, 0])) for a in range(_A)]
_CY1 = [float(np.float32(_A0[a, 1])) for a in range(_A)]
_CX2P = [float(np.float32(_A0[a, 2] + 1.0)) for a in range(_A)]
_CY2P = [float(np.float32(_A0[a, 3] + 1.0)) for a in range(_A)]
_CAREA = [float(np.float32((_A0[a, 2] - _A0[a, 0] + 1.0) *
                           (_A0[a, 3] - _A0[a, 1] + 1.0))) for a in range(_A)]
_CKLO = [int(np.ceil(-_A0[a, 0] / 16.0)) for a in range(_A)]
_RKLO = [int(np.ceil(-_A0[a, 1] / 16.0)) for a in range(_A)]

_REAL = [b * _KP + k for b in range(_B) for k in range(_KGT)]
_PADS = [b * _KP + k for b in range(_B) for k in range(_KGT, _KP)]
_SCHED = [[] for _ in range(_NW)]
for _i, _s in enumerate(_REAL):
    _SCHED[_i % _NW].append(_s)
_pit = iter(_PADS)
for _wl in _SCHED:
    while len(_wl) < _GPW:
        _wl.append(next(_pit))
_PERM = np.array([s for wl in _SCHED for s in wl], dtype=np.int64)
_INVPERM = np.empty(_NG, dtype=np.int64)
_INVPERM[_PERM] = np.arange(_NG)


def _run(gtb, lim, rpnflat, cc2, sv16, sp16, boff, gtflat):
    mesh = plsc.VectorSubcoreMesh(core_axis_name="c", subcore_axis_name="s")

    @functools.partial(
        pl.kernel,
        out_type=jax.ShapeDtypeStruct((_NG * 5,), jnp.float32),
        mesh=mesh,
        scratch_types=[
            pltpu.VMEM((5, _GPW, 16), jnp.float32),
            pltpu.VMEM((32,), jnp.float32),
            pltpu.VMEM((5, 16), jnp.float32),
            pltpu.VMEM((5, 16), jnp.float32),
            pltpu.VMEM((5, 16), jnp.float32),
            pltpu.VMEM((16,), jnp.int32),
            pltpu.VMEM((80,), jnp.float32),
            pltpu.VMEM((16,), jnp.float32),
            pltpu.VMEM((_GPW * 5,), jnp.float32),
            pltpu.SemaphoreType.DMA,
        ],
        compiler_params=pltpu.CompilerParams(use_tc_tiling_on_sc=False,
                                             needs_layout_passes=False),
    )
    def k(gtb_h, lim_h, rpn_h, cc_h, sv_h, sp_h, boff_h, gtf_h, out_h,
          gt_v, lim_v, ccv, svv, spv, bv, gtv, pbuf, obuf, sem):
        wid = lax.axis_index("s") * 2 + lax.axis_index("c")
        sbase = wid * _GPW
        pltpu.sync_copy(gtb_h.at[:, pl.ds(sbase, _GPW), :], gt_v)
        pltpu.sync_copy(lim_h, lim_v)
        for r in range(5):
            pltpu.sync_copy(cc_h.at[r, pl.ds(sbase, 16)], ccv.at[r])
        pltpu.sync_copy(sv_h, svv)
        pltpu.sync_copy(sp_h, spv)
        pltpu.sync_copy(boff_h.at[pl.ds(sbase, 16)], bv)
        pltpu.sync_copy(gtf_h.at[pl.ds(sbase * 5, _GPW * 5)], gtv.at[pl.ds(0, _GPW * 5)])

        wp_s = lax.reduce_max(lim_v[pl.ds(0, 16)], (0,))
        hp_s = lax.reduce_max(lim_v[pl.ds(16, 16)], (0,))
        inv16s = jnp.float32(0.0625)
        ckhi_l = [((wp_s - jnp.float32(_CX2P[a] + 1.0)) * inv16s)
                  .astype(jnp.int32) for a in range(_A)]
        rkhi_l = [((hp_s - jnp.float32(_CY2P[a] + 1.0)) * inv16s)
                  .astype(jnp.int32) for a in range(_A)]
        iota_i = lax.iota(jnp.int32, 16)
        iota_f = iota_i.astype(jnp.float32)
        i16f = iota_f * jnp.float32(16.0)
        i9 = iota_i * 9
        zerov = jnp.zeros((16,), jnp.float32)
        imaxv = jnp.full((16,), jnp.int32(2**31 - 1), jnp.int32)
        f16 = jnp.float32(16.0)
        inv16 = jnp.float32(0.0625)

        def per_gt(t, argacc):
            g1 = gt_v[0, t, :]
            g2 = gt_v[1, t, :]
            g3 = gt_v[2, t, :]
            g4 = gt_v[3, t, :]
            g5 = gt_v[4, t, :]
            sx1 = lax.reduce_max(g1, (0,))
            sy1 = lax.reduce_max(g2, (0,))
            sx2p = lax.reduce_max(g3, (0,))
            sy2p = lax.reduce_max(g4, (0,))

            best = jnp.full((16,), -1.0, jnp.float32)
            barg = jnp.zeros((16,), jnp.int32)
            for a in range(_A):
                cx1 = jnp.float32(_CX1[a])
                cy1 = jnp.float32(_CY1[a])
                cx2p = jnp.float32(_CX2P[a])
                cy2p = jnp.float32(_CY2P[a])
                dxw = jnp.full((16,), jnp.float32(_CX2P[a] - _CX1[a]),
                               jnp.float32)
                areag5 = jnp.full((16,), jnp.float32(_CAREA[a]),
                                  jnp.float32) + g5
                wlo = jnp.maximum(((sx1 - cx2p) * inv16).astype(jnp.int32),
                                  _CKLO[a])
                whi = jnp.minimum(
                    ((sx2p - cx1) * inv16).astype(jnp.int32) + 1, ckhi_l[a])
                hlo = jnp.maximum(((sy1 - cy2p) * inv16).astype(jnp.int32),
                                  _RKLO[a])
                hhi = jnp.minimum(
                    ((sy2p - cy1) * inv16).astype(jnp.int32) + 1, rkhi_l[a])
                ncol = lax.shift_right_logical(jnp.maximum(whi - wlo, 0), 4) + 1
                whiv = jax.lax.broadcast(whi, (16,))

                def row_body(h, carry2):
                    hf = h.astype(jnp.float32)
                    ay1 = hf * f16 + cy1
                    ay2p = hf * f16 + cy2p
                    ihs = jnp.maximum(
                        jnp.minimum(ay2p, sy2p) - jnp.maximum(ay1, sy1),
                        jnp.float32(0.0))
                    ihbc = jax.lax.broadcast(ihs, (16,))
                    rowb9 = h * 1080 + a

                    def col_body(j, carry3):
                        bb, ba = carry3
                        col0 = wlo + j * 16
                        c0f = col0.astype(jnp.float32)
                        x1v = jax.lax.broadcast(c0f * f16 + cx1, (16,)) + i16f
                        x2pv = x1v + dxw
                        iw = jnp.minimum(x2pv, g3) - jnp.maximum(x1v, g1)
                        iwc = jnp.maximum(iw, zerov)
                        inter = iwc * ihbc
                        ua = areag5 - inter
                        v = inter / ua
                        coliv = jax.lax.broadcast(col0, (16,)) + iota_i
                        v = jnp.where(coliv <= whiv, v, zerov)
                        idxv = jax.lax.broadcast(col0 * 9 + rowb9, (16,)) + i9
                        gtm = v > bb
                        upd = gtm | ((v == bb) & (idxv < ba))
                        bb = jnp.where(gtm, v, bb)
                        ba = jnp.where(upd, idxv, ba)
                        return bb, ba

                    return lax.fori_loop(0, ncol, col_body, carry2)

                best, barg = lax.fori_loop(hlo, hhi + 1, row_body,
                                           (best, barg))

            m = lax.reduce_max(best, (0,))
            eq = best == jax.lax.broadcast(m, (16,))
            cand = jnp.where(eq, barg, imaxv)
            mi = lax.reduce_min(cand, (0,))
            argacc = jnp.where(iota_i == jax.lax.broadcast(t, (16,)),
                               jax.lax.broadcast(mi, (16,)), argacc)
            return argacc

        argacc = lax.fori_loop(0, _GPW, per_gt, jnp.zeros((16,), jnp.int32))

        gidx = jnp.minimum(argacc + bv[...], jnp.int32(_NRPN - 1))
        pltpu.async_copy(rpn_h.at[gidx], pbuf, sem).wait()
        p = pbuf[...]
        one = jnp.ones((16,), jnp.float32)
        aprod = jnp.ones((16,), jnp.float32)
        bprod = jnp.ones((16,), jnp.float32)
        for r in range(5):
            ccr = ccv[r, :]
            s = svv[r, :]
            sp = spv[r, :]
            hit = ccr == one
            aprod = aprod * jnp.where(hit, s, one - s)
            bprod = bprod * jnp.where(hit, one - sp, sp)
        ap = aprod * p
        bq = bprod * (one - p)
        mu = ap / ((ap + bq) + jnp.float32(1e-12))
        bg = mu < jnp.float32(0.5)
        lanemask = iota_i < _GPW
        for c in range(5):
            idxc = iota_i * 5 + c
            gvals = plsc.load_gather(gtv, [idxc], mask=lanemask)
            plsc.store_scatter(obuf, [idxc], jnp.where(bg, zerov, gvals),
                               mask=lanemask)
        pltpu.sync_copy(obuf, out_h.at[pl.ds(sbase * 5, _GPW * 5)])

    return k(gtb, lim, rpnflat, cc2, sv16, sp16, boff, gtflat)


def kernel(rpn_cls_prob, gt_boxes, num_boxes, im_info, crowdsourced_classes,
           alpha_con):
    del num_boxes
    gtp = jnp.pad(gt_boxes, ((0, 0), (0, _KP - _KGT), (0, 0)))
    gtp = gtp.at[:, _KGT:, 0:2].set(1.0e6)
    gtp = gtp.at[:, _KGT:, 2:4].set(1.0e6 + 19.0)
    gtp = gtp.reshape(_NG, 5)[_PERM]
    gx1 = gtp[:, 0]
    gy1 = gtp[:, 1]
    gx2p = gtp[:, 2] + 1.0
    gy2p = gtp[:, 3] + 1.0
    garea = (gtp[:, 2] - gtp[:, 0] + 1.0) * \
            (gtp[:, 3] - gtp[:, 1] + 1.0)
    gtb = jnp.stack([gx1, gy1, gx2p, gy2p, garea])
    gtb = jnp.broadcast_to(gtb[:, :, None], (5, _NG, 16))
    gtb = jnp.asarray(gtb, jnp.float32)

    wp = im_info[0, 1] + 1.0
    hp = im_info[0, 0] + 1.0
    lim = jnp.concatenate([jnp.full((16,), wp, jnp.float32),
                           jnp.full((16,), hp, jnp.float32)])

    rpnflat = rpn_cls_prob.reshape(-1)
    ccp = jnp.pad(crowdsourced_classes, ((0, 0), (0, _KP - _KGT), (0, 0)))
    cc2 = jnp.transpose(ccp, (2, 0, 1)).reshape(_R, _NG)[:, _PERM]
    cc2 = jnp.pad(cc2, ((0, 0), (0, 16)))
    asum = jnp.sum(alpha_con, axis=2, keepdims=True)
    alpha = alpha_con / asum
    sens = alpha[:, 1, 1]
    spec = alpha[:, 0, 0]
    sv16 = jnp.broadcast_to(sens[:, None], (_R, 16))
    sp16 = jnp.broadcast_to(spec[:, None], (_R, 16))
    boff = jnp.repeat(
        jnp.arange(_B, dtype=jnp.int32) * (2 * _A * _FEAT_H * _FEAT_W) +
        jnp.int32(_A * _FEAT_H * _FEAT_W), _KP)[_PERM]
    boff = jnp.pad(boff, (0, 16))
    gtflat = gtp.reshape(-1)

    outflat = _run(gtb, lim, rpnflat, cc2, sv16, sp16, boff, gtflat)
    out = outflat.reshape(_NG, 5)[_INVPERM]
    return out.reshape(_B, _KP, 5)[:, :_KGT, :]


# speedup vs baseline: 1.6464x; 1.6313x over previous
import functools
import numpy as np
import jax
import jax.numpy as jnp
from jax import lax
from jax.experimental import pallas as pl
from jax.experimental.pallas import tpu as pltpu
from jax.experimental.pallas import tpu_sc as plsc

FEAT_STRIDE = 16
_SCALES = np.array([8.0, 16.0, 32.0])
_RATIOS = np.array([0.5, 1.0, 2.0])


def _gen_anchors_np():
    base = np.array([1, 1, 16, 16], dtype=np.float64) - 1
    w = base[2] - base[0] + 1
    h = base[3] - base[1] + 1
    x_ctr = base[0] + 0.5 * (w - 1)
    y_ctr = base[1] + 0.5 * (h - 1)
    size = w * h
    size_ratios = size / _RATIOS
    ws0 = np.round(np.sqrt(size_ratios))
    hs0 = np.round(ws0 * _RATIOS)
    ratio_anchors = np.hstack((
        x_ctr - 0.5 * (ws0[:, None] - 1), y_ctr - 0.5 * (hs0[:, None] - 1),
        x_ctr + 0.5 * (ws0[:, None] - 1), y_ctr + 0.5 * (hs0[:, None] - 1)))
    outs = []
    for i in range(ratio_anchors.shape[0]):
        a = ratio_anchors[i]
        aw = a[2] - a[0] + 1
        ah = a[3] - a[1] + 1
        ax = a[0] + 0.5 * (aw - 1)
        ay = a[1] + 0.5 * (ah - 1)
        ws = aw * _SCALES
        hs = ah * _SCALES
        outs.append(np.hstack((
            ax - 0.5 * (ws[:, None] - 1), ay - 0.5 * (hs[:, None] - 1),
            ax + 0.5 * (ws[:, None] - 1), ay + 0.5 * (hs[:, None] - 1))))
    return np.vstack(outs)


_FEAT_H, _FEAT_W = 75, 120
_A = 9
_N_TRUE = _FEAT_H * _FEAT_W * _A
_NW = 32
_B, _KGT, _R = 4, 50, 5
_KP = 64
_NG = _B * _KP
_GPW = _NG // _NW
_NRPN = _B * 2 * _A * _FEAT_H * _FEAT_W

_A0 = _gen_anchors_np()
_CX1 = [float(np.float32(_A0[a, 0])) for a in range(_A)]
_CY1 = [float(np.float32(_A0[a, 1])) for a in range(_A)]
_CX2P = [float(np.float32(_A0[a, 2] + 1.0)) for a in range(_A)]
_CY2P = [float(np.float32(_A0[a, 3] + 1.0)) for a in range(_A)]
_CAREA = [float(np.float32((_A0[a, 2] - _A0[a, 0] + 1.0) *
                           (_A0[a, 3] - _A0[a, 1] + 1.0))) for a in range(_A)]
_CKLO = [int(np.ceil(-_A0[a, 0] / 16.0)) for a in range(_A)]
_RKLO = [int(np.ceil(-_A0[a, 1] / 16.0)) for a in range(_A)]

_REAL = [b * _KP + k for b in range(_B) for k in range(_KGT)]
_PADS = [b * _KP + k for b in range(_B) for k in range(_KGT, _KP)]
_SCHED = [[] for _ in range(_NW)]
for _i, _s in enumerate(_REAL):
    _SCHED[_i % _NW].append(_s)
_pit = iter(_PADS)
for _wl in _SCHED:
    while len(_wl) < _GPW:
        _wl.append(next(_pit))
_PERM = np.array([s for wl in _SCHED for s in wl], dtype=np.int64)
_INVPERM = np.empty(_NG, dtype=np.int64)
_INVPERM[_PERM] = np.arange(_NG)


def _run(gtb, lim, rpnflat, cc2, sv16, sp16, boff, gtflat):
    mesh = plsc.VectorSubcoreMesh(core_axis_name="c", subcore_axis_name="s")

    @functools.partial(
        pl.kernel,
        out_type=jax.ShapeDtypeStruct((_NG * 5,), jnp.float32),
        mesh=mesh,
        scratch_types=[
            pltpu.VMEM((5, _GPW, 16), jnp.float32),
            pltpu.VMEM((32,), jnp.float32),
            pltpu.VMEM((5, 16), jnp.float32),
            pltpu.VMEM((5, 16), jnp.float32),
            pltpu.VMEM((5, 16), jnp.float32),
            pltpu.VMEM((16,), jnp.int32),
            pltpu.VMEM((80,), jnp.float32),
            pltpu.VMEM((16,), jnp.float32),
            pltpu.VMEM((_GPW * 5,), jnp.float32),
            pltpu.SemaphoreType.DMA,
        ],
        compiler_params=pltpu.CompilerParams(use_tc_tiling_on_sc=False,
                                             needs_layout_passes=False),
    )
    def k(gtb_h, lim_h, rpn_h, cc_h, sv_h, sp_h, boff_h, gtf_h, out_h,
          gt_v, lim_v, ccv, svv, spv, bv, gtv, pbuf, obuf, sem):
        wid = lax.axis_index("s") * 2 + lax.axis_index("c")
        sbase = wid * _GPW
        pltpu.sync_copy(gtb_h.at[:, pl.ds(sbase, _GPW), :], gt_v)
        pltpu.sync_copy(lim_h, lim_v)
        for r in range(5):
            pltpu.sync_copy(cc_h.at[r, pl.ds(sbase, 16)], ccv.at[r])
        pltpu.sync_copy(sv_h, svv)
        pltpu.sync_copy(sp_h, spv)
        pltpu.sync_copy(boff_h.at[pl.ds(sbase, 16)], bv)
        pltpu.sync_copy(gtf_h.at[pl.ds(sbase * 5, _GPW * 5)], gtv.at[pl.ds(0, _GPW * 5)])

        wp_s = lax.reduce_max(lim_v[pl.ds(0, 16)], (0,))
        hp_s = lax.reduce_max(lim_v[pl.ds(16, 16)], (0,))
        inv16s = jnp.float32(0.0625)
        ckhi_l = [((wp_s - jnp.float32(_CX2P[a] + 1.0)) * inv16s)
                  .astype(jnp.int32) for a in range(_A)]
        rkhi_l = [((hp_s - jnp.float32(_CY2P[a] + 1.0)) * inv16s)
                  .astype(jnp.int32) for a in range(_A)]
        iota_i = lax.iota(jnp.int32, 16)
        iota_f = iota_i.astype(jnp.float32)
        i16f = iota_f * jnp.float32(16.0)
        i9 = iota_i * 9
        zerov = jnp.zeros((16,), jnp.float32)
        imaxv = jnp.full((16,), jnp.int32(2**31 - 1), jnp.int32)
        f16 = jnp.float32(16.0)
        inv16 = jnp.float32(0.0625)

        def per_gt(t, argacc):
            g1 = gt_v[0, t, :]
            g2 = gt_v[1, t, :]
            g3 = gt_v[2, t, :]
            g4 = gt_v[3, t, :]
            g5 = gt_v[4, t, :]
            sx1 = lax.reduce_max(g1, (0,))
            sy1 = lax.reduce_max(g2, (0,))
            sx2p = lax.reduce_max(g3, (0,))
            sy2p = lax.reduce_max(g4, (0,))

            best = jnp.full((16,), -1.0, jnp.float32)
            barg = jnp.zeros((16,), jnp.int32)
            for a in range(_A):
                cx1 = jnp.float32(_CX1[a])
                cy1 = jnp.float32(_CY1[a])
                cx2p = jnp.float32(_CX2P[a])
                cy2p = jnp.float32(_CY2P[a])
                dxw = jnp.full((16,), jnp.float32(_CX2P[a] - _CX1[a]),
                               jnp.float32)
                areag5 = jnp.full((16,), jnp.float32(_CAREA[a]),
                                  jnp.float32) + g5
                wlo = jnp.maximum(((sx1 - cx2p) * inv16).astype(jnp.int32),
                                  _CKLO[a])
                whi = jnp.minimum(
                    ((sx2p - cx1) * inv16).astype(jnp.int32) + 1, ckhi_l[a])
                hlo = jnp.maximum(((sy1 - cy2p) * inv16).astype(jnp.int32),
                                  _RKLO[a])
                hhi = jnp.minimum(
                    ((sy2p - cy1) * inv16).astype(jnp.int32) + 1, rkhi_l[a])
                ncol = lax.shift_right_logical(jnp.maximum(whi - wlo, 0), 4) + 1
                whiv = jax.lax.broadcast(whi, (16,))

                def chunk_body(j, carry2):
                    col0 = wlo + j * 16
                    c0f = col0.astype(jnp.float32)
                    x1v = jax.lax.broadcast(c0f * f16 + cx1, (16,)) + i16f
                    x2pv = x1v + dxw
                    iw = jnp.minimum(x2pv, g3) - jnp.maximum(x1v, g1)
                    iwc = jnp.maximum(iw, zerov)
                    coliv = jax.lax.broadcast(col0, (16,)) + iota_i
                    iwm = jnp.where(coliv <= whiv, iwc, zerov)
                    ci9v = coliv * 9 + a

                    def row_body(h, carry3):
                        bb, ba = carry3
                        hf = h.astype(jnp.float32)
                        ay1 = hf * f16 + cy1
                        ay2p = hf * f16 + cy2p
                        ihs = jnp.maximum(
                            jnp.minimum(ay2p, sy2p) - jnp.maximum(ay1, sy1),
                            jnp.float32(0.0))
                        inter = iwm * jax.lax.broadcast(ihs, (16,))
                        ua = areag5 - inter
                        v = inter / ua
                        idxv = ci9v + jax.lax.broadcast(h * 1080, (16,))
                        gtm = v > bb
                        upd = gtm | ((v == bb) & (idxv < ba))
                        bb = jnp.where(gtm, v, bb)
                        ba = jnp.where(upd, idxv, ba)
                        return bb, ba

                    return lax.fori_loop(hlo, hhi + 1, row_body, carry2)

                best, barg = lax.fori_loop(0, ncol, chunk_body,
                                           (best, barg))

            m = lax.reduce_max(best, (0,))
            eq = best == jax.lax.broadcast(m, (16,))
            cand = jnp.where(eq, barg, imaxv)
            mi = lax.reduce_min(cand, (0,))
            argacc = jnp.where(iota_i == jax.lax.broadcast(t, (16,)),
                               jax.lax.broadcast(mi, (16,)), argacc)
            return argacc

        argacc = lax.fori_loop(0, _GPW, per_gt, jnp.zeros((16,), jnp.int32))

        gidx = jnp.minimum(argacc + bv[...], jnp.int32(_NRPN - 1))
        pltpu.async_copy(rpn_h.at[gidx], pbuf, sem).wait()
        p = pbuf[...]
        one = jnp.ones((16,), jnp.float32)
        aprod = jnp.ones((16,), jnp.float32)
        bprod = jnp.ones((16,), jnp.float32)
        for r in range(5):
            ccr = ccv[r, :]
            s = svv[r, :]
            sp = spv[r, :]
            hit = ccr == one
            aprod = aprod * jnp.where(hit, s, one - s)
            bprod = bprod * jnp.where(hit, one - sp, sp)
        ap = aprod * p
        bq = bprod * (one - p)
        mu = ap / ((ap + bq) + jnp.float32(1e-12))
        bg = mu < jnp.float32(0.5)
        lanemask = iota_i < _GPW
        for c in range(5):
            idxc = iota_i * 5 + c
            gvals = plsc.load_gather(gtv, [idxc], mask=lanemask)
            plsc.store_scatter(obuf, [idxc], jnp.where(bg, zerov, gvals),
                               mask=lanemask)
        pltpu.sync_copy(obuf, out_h.at[pl.ds(sbase * 5, _GPW * 5)])

    return k(gtb, lim, rpnflat, cc2, sv16, sp16, boff, gtflat)


def kernel(rpn_cls_prob, gt_boxes, num_boxes, im_info, crowdsourced_classes,
           alpha_con):
    del num_boxes
    gtp = jnp.pad(gt_boxes, ((0, 0), (0, _KP - _KGT), (0, 0)))
    gtp = gtp.at[:, _KGT:, 0:2].set(1.0e6)
    gtp = gtp.at[:, _KGT:, 2:4].set(1.0e6 + 19.0)
    gtp = gtp.reshape(_NG, 5)[_PERM]
    gx1 = gtp[:, 0]
    gy1 = gtp[:, 1]
    gx2p = gtp[:, 2] + 1.0
    gy2p = gtp[:, 3] + 1.0
    garea = (gtp[:, 2] - gtp[:, 0] + 1.0) * \
            (gtp[:, 3] - gtp[:, 1] + 1.0)
    gtb = jnp.stack([gx1, gy1, gx2p, gy2p, garea])
    gtb = jnp.broadcast_to(gtb[:, :, None], (5, _NG, 16))
    gtb = jnp.asarray(gtb, jnp.float32)

    wp = im_info[0, 1] + 1.0
    hp = im_info[0, 0] + 1.0
    lim = jnp.concatenate([jnp.full((16,), wp, jnp.float32),
                           jnp.full((16,), hp, jnp.float32)])

    rpnflat = rpn_cls_prob.reshape(-1)
    ccp = jnp.pad(crowdsourced_classes, ((0, 0), (0, _KP - _KGT), (0, 0)))
    cc2 = jnp.transpose(ccp, (2, 0, 1)).reshape(_R, _NG)[:, _PERM]
    cc2 = jnp.pad(cc2, ((0, 0), (0, 16)))
    asum = jnp.sum(alpha_con, axis=2, keepdims=True)
    alpha = alpha_con / asum
    sens = alpha[:, 1, 1]
    spec = alpha[:, 0, 0]
    sv16 = jnp.broadcast_to(sens[:, None], (_R, 16))
    sp16 = jnp.broadcast_to(spec[:, None], (_R, 16))
    boff = jnp.repeat(
        jnp.arange(_B, dtype=jnp.int32) * (2 * _A * _FEAT_H * _FEAT_W) +
        jnp.int32(_A * _FEAT_H * _FEAT_W), _KP)[_PERM]
    boff = jnp.pad(boff, (0, 16))
    gtflat = gtp.reshape(-1)

    outflat = _run(gtb, lim, rpnflat, cc2, sv16, sp16, boff, gtflat)
    out = outflat.reshape(_NG, 5)[_INVPERM]
    return out.reshape(_B, _KP, 5)[:, :_KGT, :]
